# baseline (speedup 1.0000x reference)
import jax
import jax.numpy as jnp
from jax import lax
from jax.experimental import pallas as pl
from jax.experimental.pallas import tpu as pltpu
from jax.experimental.pallas import tpu_sc as plsc

BATCH = 16384
EMBED_K = 32
ROW_PACK = 4
SUPER_W = EMBED_K * ROW_PACK
NUM_CORES = 2
NUM_SUBCORES = 16
NUM_WORKERS = NUM_CORES * NUM_SUBCORES
PER_WORKER = BATCH // NUM_WORKERS
CHUNK = 128
NUM_CHUNKS = PER_WORKER // CHUNK
PASS_ROWS = 256
NUM_PASSES = PER_WORKER // PASS_ROWS
LANES = 16


def _sc_body(w_hbm, h_hbm, u_hbm, v_hbm, out_hbm,
             u_idx, v_idx, u_sup, v_sup, u_off, v_off,
             u_rows, v_rows, out_v, sem):
    wid = lax.axis_index("c") * NUM_SUBCORES + lax.axis_index("s")
    base = wid * PER_WORKER

    pltpu.sync_copy(u_hbm.at[wid], u_idx)
    pltpu.sync_copy(v_hbm.at[wid], v_idx)

    for c in range(NUM_CHUNKS):
        for i in range(CHUNK // LANES):
            sl = pl.ds(i * LANES, LANES)
            ui = u_idx[c, sl]
            vi = v_idx[c, sl]
            u_sup[c, sl] = lax.shift_right_logical(ui, 2)
            v_sup[c, sl] = lax.shift_right_logical(vi, 2)
            osl = pl.ds((c * CHUNK + i * LANES), LANES)
            u_off[osl] = lax.shift_left(lax.bitwise_and(ui, 3), 5)
            v_off[osl] = lax.shift_left(lax.bitwise_and(vi, 3), 5)

    lane = lax.iota(jnp.int32, LANES)

    for p in range(NUM_PASSES):
        copies = []
        for c in range(PASS_ROWS // CHUNK):
            rows_sl = pl.ds(c * CHUNK, CHUNK)
            sup_row = p * (PASS_ROWS // CHUNK) + c
            copies.append(pltpu.async_copy(w_hbm.at[u_sup.at[sup_row]],
                                           u_rows.at[rows_sl], sem))
            copies.append(pltpu.async_copy(h_hbm.at[v_sup.at[sup_row]],
                                           v_rows.at[rows_sl], sem))
        for cp in copies:
            cp.wait()

        def group(j, carry):
            rows = lane + j * LANES
            gsl = pl.ds(pl.multiple_of(p * PASS_ROWS + j * LANES, LANES),
                        LANES)
            ucol = u_off[gsl]
            vcol = v_off[gsl]
            acc = jnp.zeros((LANES,), jnp.float32)
            for k in range(EMBED_K):
                uk = plsc.load_gather(u_rows, [rows, ucol + k])
                vk = plsc.load_gather(v_rows, [rows, vcol + k])
                acc = acc + uk * vk
            out_v[gsl] = 1.0 / (1.0 + jnp.exp(-acc))
            return carry

        lax.fori_loop(0, PASS_ROWS // LANES, group, 0, unroll=2)

    pltpu.sync_copy(out_v, out_hbm.at[pl.ds(base, PER_WORKER)])


@jax.jit
def kernel(x, W, H):
    u = x[:, 0].astype(jnp.int32).reshape(NUM_WORKERS, NUM_CHUNKS, CHUNK)
    v = x[:, 1].astype(jnp.int32).reshape(NUM_WORKERS, NUM_CHUNKS, CHUNK)
    w_sup = W.reshape(-1, SUPER_W)
    h_sup = H.reshape(-1, SUPER_W)
    mesh = plsc.VectorSubcoreMesh(core_axis_name="c", subcore_axis_name="s")
    run = pl.kernel(
        _sc_body,
        out_type=jax.ShapeDtypeStruct((BATCH,), jnp.float32),
        mesh=mesh,
        scratch_types=[
            pltpu.VMEM((NUM_CHUNKS, CHUNK), jnp.int32),
            pltpu.VMEM((NUM_CHUNKS, CHUNK), jnp.int32),
            pltpu.VMEM((NUM_CHUNKS, CHUNK), jnp.int32),
            pltpu.VMEM((NUM_CHUNKS, CHUNK), jnp.int32),
            pltpu.VMEM((PER_WORKER,), jnp.int32),
            pltpu.VMEM((PER_WORKER,), jnp.int32),
            pltpu.VMEM((PASS_ROWS, SUPER_W), jnp.float32),
            pltpu.VMEM((PASS_ROWS, SUPER_W), jnp.float32),
            pltpu.VMEM((PER_WORKER,), jnp.float32),
            pltpu.SemaphoreType.DMA,
        ],
        compiler_params=pltpu.CompilerParams(needs_layout_passes=False),
    )
    return run(w_sup, h_sup, u, v)


# speedup vs baseline: 2.9595x; 2.9595x over previous
import jax
import jax.numpy as jnp
from jax import lax
from jax.experimental import pallas as pl
from jax.experimental.pallas import tpu as pltpu
from jax.experimental.pallas import tpu_sc as plsc

BATCH = 16384
EMBED_K = 32
NUM_CORES = 2
NUM_SUBCORES = 16
NUM_WORKERS = NUM_CORES * NUM_SUBCORES
PER_WORKER = BATCH // NUM_WORKERS
CHUNK = 128
NUM_CHUNKS = PER_WORKER // CHUNK
LANES = 16
NUM_GROUPS = PER_WORKER // LANES
RING = 2


def _sc_body(w_hbm, h_hbm, u_hbm, v_hbm, out_hbm,
             u_vm, v_vm, ru, rv, pp, out_v, sem):
    wid = lax.axis_index("c") * NUM_SUBCORES + lax.axis_index("s")
    base = wid * PER_WORKER

    pltpu.sync_copy(u_hbm.at[wid], u_vm.at[pl.ds(0, PER_WORKER)])
    pltpu.sync_copy(v_hbm.at[wid], v_vm.at[pl.ds(0, PER_WORKER)])

    kidx = lax.iota(jnp.int32, LANES)

    def fire(p, slot):
        uu = u_vm[pl.ds(p, LANES)][0]
        vv = v_vm[pl.ds(p, LANES)][0]
        ub = lax.shift_left(lax.shift_right_logical(uu, 7), 7)
        vb = lax.shift_left(lax.shift_right_logical(vv, 7), 7)
        pltpu.async_copy(
            w_hbm.at[:, pl.ds(pl.multiple_of(ub, CHUNK), CHUNK)],
            ru.at[slot], sem)
        pltpu.async_copy(
            h_hbm.at[:, pl.ds(pl.multiple_of(vb, CHUNK), CHUNK)],
            rv.at[slot], sem)

    def drain_one(slot):
        pltpu.make_async_copy(w_hbm.at[:, pl.ds(0, CHUNK)],
                              ru.at[slot], sem).wait()
        pltpu.make_async_copy(h_hbm.at[:, pl.ds(0, CHUNK)],
                              rv.at[slot], sem).wait()

    for p in range(RING):
        fire(p, p)

    def step(p, carry):
        slot = lax.rem(p, RING)
        uu = u_vm[pl.ds(p, LANES)][0]
        vv = v_vm[pl.ds(p, LANES)][0]
        ucol = jnp.full((LANES,), lax.bitwise_and(uu, CHUNK - 1), jnp.int32)
        vcol = jnp.full((LANES,), lax.bitwise_and(vv, CHUNK - 1), jnp.int32)
        sfull = jnp.full((LANES,), slot, jnp.int32)
        drain_one(slot)
        ulo = plsc.load_gather(ru, [sfull, kidx, ucol])
        uhi = plsc.load_gather(ru, [sfull, kidx + LANES, ucol])
        vlo = plsc.load_gather(rv, [sfull, kidx, vcol])
        vhi = plsc.load_gather(rv, [sfull, kidx + LANES, vcol])

        @pl.when(p < PER_WORKER - RING)
        def _():
            fire(p + RING, slot)

        pp[p, pl.ds(0, LANES)] = ulo * vlo + uhi * vhi
        return carry

    lax.fori_loop(0, PER_WORKER, step, 0)

    lane = lax.iota(jnp.int32, LANES)

    def group(g, carry):
        rows = lane + g * LANES
        acc = jnp.zeros((LANES,), jnp.float32)
        for j in range(LANES):
            col = jnp.full((LANES,), j, jnp.int32)
            acc = acc + plsc.load_gather(pp, [rows, col])
        res = 1.0 / (1.0 + jnp.exp(-acc))
        out_v[pl.ds(pl.multiple_of(g * LANES, LANES), LANES)] = res
        return carry

    lax.fori_loop(0, NUM_GROUPS, group, 0, unroll=2)

    pltpu.sync_copy(out_v, out_hbm.at[pl.ds(base, PER_WORKER)])


@jax.jit
def kernel(x, W, H):
    u = x[:, 0].astype(jnp.int32).reshape(NUM_WORKERS, PER_WORKER)
    v = x[:, 1].astype(jnp.int32).reshape(NUM_WORKERS, PER_WORKER)
    mesh = plsc.VectorSubcoreMesh(core_axis_name="c", subcore_axis_name="s")
    run = pl.kernel(
        _sc_body,
        out_type=jax.ShapeDtypeStruct((BATCH,), jnp.float32),
        mesh=mesh,
        scratch_types=[
            pltpu.VMEM((PER_WORKER + LANES,), jnp.int32),
            pltpu.VMEM((PER_WORKER + LANES,), jnp.int32),
            pltpu.VMEM((RING, EMBED_K, CHUNK), jnp.float32),
            pltpu.VMEM((RING, EMBED_K, CHUNK), jnp.float32),
            pltpu.VMEM((PER_WORKER, LANES), jnp.float32),
            pltpu.VMEM((PER_WORKER,), jnp.float32),
            pltpu.SemaphoreType.DMA,
        ],
        compiler_params=pltpu.CompilerParams(needs_layout_passes=False),
    )
    return run(W.T, H.T, u, v)


# speedup vs baseline: 4.5468x; 1.5363x over previous
import jax
import jax.numpy as jnp
from jax import lax
from jax.experimental import pallas as pl
from jax.experimental.pallas import tpu as pltpu
from jax.experimental.pallas import tpu_sc as plsc

BATCH = 16384
EMBED_K = 32
NUM_CORES = 2
NUM_SUBCORES = 16
NUM_WORKERS = NUM_CORES * NUM_SUBCORES
PER_WORKER = BATCH // NUM_WORKERS
CHUNK = 128
NUM_CHUNKS = PER_WORKER // CHUNK
LANES = 16
NUM_GROUPS = PER_WORKER // LANES
RING = 6


def _sc_body(w_hbm, h_hbm, u_hbm, v_hbm, out_hbm,
             u_vm, v_vm, ru, rv, pp, out_v, sem):
    wid = lax.axis_index("c") * NUM_SUBCORES + lax.axis_index("s")
    base = wid * PER_WORKER

    pltpu.sync_copy(u_hbm.at[wid], u_vm.at[pl.ds(0, PER_WORKER)])
    pltpu.sync_copy(v_hbm.at[wid], v_vm.at[pl.ds(0, PER_WORKER)])

    kidx = lax.iota(jnp.int32, LANES)

    def fire(p, slot):
        uu = u_vm[pl.ds(p, LANES)][0]
        vv = v_vm[pl.ds(p, LANES)][0]
        ub = lax.shift_left(lax.shift_right_logical(uu, 7), 7)
        vb = lax.shift_left(lax.shift_right_logical(vv, 7), 7)
        pltpu.async_copy(
            w_hbm.at[:, pl.ds(pl.multiple_of(ub, CHUNK), CHUNK)],
            ru.at[slot], sem)
        pltpu.async_copy(
            h_hbm.at[:, pl.ds(pl.multiple_of(vb, CHUNK), CHUNK)],
            rv.at[slot], sem)

    def drain_one(slot):
        pltpu.make_async_copy(w_hbm.at[:, pl.ds(0, CHUNK)],
                              ru.at[slot], sem).wait()
        pltpu.make_async_copy(h_hbm.at[:, pl.ds(0, CHUNK)],
                              rv.at[slot], sem).wait()

    for p in range(RING):
        fire(p, p)

    def step(p, carry):
        slot = lax.rem(p, RING)
        uu = u_vm[pl.ds(p, LANES)][0]
        vv = v_vm[pl.ds(p, LANES)][0]
        ucol = jnp.full((LANES,), lax.bitwise_and(uu, CHUNK - 1), jnp.int32)
        vcol = jnp.full((LANES,), lax.bitwise_and(vv, CHUNK - 1), jnp.int32)
        sfull = jnp.full((LANES,), slot, jnp.int32)
        drain_one(slot)
        ulo = plsc.load_gather(ru, [sfull, kidx, ucol])
        uhi = plsc.load_gather(ru, [sfull, kidx + LANES, ucol])
        vlo = plsc.load_gather(rv, [sfull, kidx, vcol])
        vhi = plsc.load_gather(rv, [sfull, kidx + LANES, vcol])

        @pl.when(p < PER_WORKER - RING)
        def _():
            fire(p + RING, slot)

        pp[p, pl.ds(0, LANES)] = ulo * vlo + uhi * vhi
        return carry

    lax.fori_loop(0, PER_WORKER, step, 0)

    lane = lax.iota(jnp.int32, LANES)

    def group(g, carry):
        rows = lane + g * LANES
        acc = jnp.zeros((LANES,), jnp.float32)
        for j in range(LANES):
            col = jnp.full((LANES,), j, jnp.int32)
            acc = acc + plsc.load_gather(pp, [rows, col])
        res = 1.0 / (1.0 + jnp.exp(-acc))
        out_v[pl.ds(pl.multiple_of(g * LANES, LANES), LANES)] = res
        return carry

    lax.fori_loop(0, NUM_GROUPS, group, 0, unroll=2)

    pltpu.sync_copy(out_v, out_hbm.at[pl.ds(base, PER_WORKER)])


@jax.jit
def kernel(x, W, H):
    u = x[:, 0].astype(jnp.int32).reshape(NUM_WORKERS, PER_WORKER)
    v = x[:, 1].astype(jnp.int32).reshape(NUM_WORKERS, PER_WORKER)
    mesh = plsc.VectorSubcoreMesh(core_axis_name="c", subcore_axis_name="s")
    run = pl.kernel(
        _sc_body,
        out_type=jax.ShapeDtypeStruct((BATCH,), jnp.float32),
        mesh=mesh,
        scratch_types=[
            pltpu.VMEM((PER_WORKER + LANES,), jnp.int32),
            pltpu.VMEM((PER_WORKER + LANES,), jnp.int32),
            pltpu.VMEM((RING, EMBED_K, CHUNK), jnp.float32),
            pltpu.VMEM((RING, EMBED_K, CHUNK), jnp.float32),
            pltpu.VMEM((PER_WORKER, LANES), jnp.float32),
            pltpu.VMEM((PER_WORKER,), jnp.float32),
            pltpu.SemaphoreType.DMA,
        ],
        compiler_params=pltpu.CompilerParams(needs_layout_passes=False),
    )
    return run(W.T, H.T, u, v)
